# baseline (speedup 1.0000x reference)
import jax
import jax.numpy as jnp
from jax import lax
from jax.experimental import pallas as pl
from jax.experimental.pallas import tpu as pltpu
from jax.experimental.pallas import tpu_sc as plsc

_B = 16
_NUM_CLASSES = 20
_NUM_ROWS = 240
_NUM_COLS = 240
_MAPB = _NUM_CLASSES * _NUM_ROWS * _NUM_COLS
_N = 1_000_000
_NP = 1_024_000
_SPAN = _NP // 16
_K = 2_000
_NCHUNK = _SPAN // _K
_SLICE = _MAPB // 16
_ZB = 7_200
_NSUB = _SLICE // _ZB
_MAGIC = 12582912.0


def _body(xs, ys, zs, bi, se, hx, hy, hz, cs, sn,
          out, gidx, gval,
          thx, thy, thz, tc, ts,
          xb0, yb0, zb0, bib0, seb0, gib0, gvb0,
          xb1, yb1, zb1, bib1, seb1, gib1, gvb1,
          buf0, buf1, smap,
          sin0, sin1, sout0, sout1, ssc0, ssc1, swb0, swb1):
    cid = lax.axis_index("c")
    sid = lax.axis_index("s")
    goff = cid * _NP + sid * _SPAN

    in_slots = ((xb0, yb0, zb0, bib0, seb0, sin0),
                (xb1, yb1, zb1, bib1, seb1, sin1))
    pair_slots = ((gib0, gvb0), (gib1, gvb1))
    out_sems = (sout0, sout1)
    sc_sems = (ssc0, ssc1)
    wb = ((buf0, swb0), (buf1, swb1))

    def a_loads(c, s, issue):
        xb, yb, zb, bib, seb, sem = in_slots[s]
        base = sid * _SPAN + c * _K
        for src, dst in ((xs, xb), (ys, yb), (zs, zb), (bi, bib), (se, seb)):
            d = pltpu.make_async_copy(src.at[pl.ds(base, _K)], dst, sem)
            d.start() if issue else d.wait()

    def a_outs(c, s, issue):
        gib, gvb = pair_slots[s]
        base = goff + c * _K
        for src, dst in ((gib, gidx.at[pl.ds(base, _K)]),
                         (gvb, gval.at[pl.ds(base, _K)])):
            d = pltpu.make_async_copy(src, dst, out_sems[s])
            d.start() if issue else d.wait()

    pltpu.sync_copy(hx, thx)
    pltpu.sync_copy(hy, thy)
    pltpu.sync_copy(hz, thz)
    pltpu.sync_copy(cs, tc)
    pltpu.sync_copy(sn, ts)
    _ABLATE_A = True
    _ABLATE_SCAN = False
    if not _ABLATE_A:
        a_loads(0, 0, True)
        a_loads(1, 1, True)

    def a_compute(s):
        xb, yb, zb, bib, seb, _ = in_slots[s]
        gib, gvb = pair_slots[s]

        def vbody(v, _):
            for u in range(5):
                sl = pl.ds((v * 5 + u) * 16, 16)
                xv = xb[sl]
                yv = yb[sl]
                zv = zb[sl]
                biv = bib[sl]
                sev = seb[sl]
                hxv = plsc.load_gather(thx, [biv])
                hyv = plsc.load_gather(thy, [biv])
                hzv = plsc.load_gather(thz, [biv])
                cv = plsc.load_gather(tc, [biv])
                sv = plsc.load_gather(ts, [biv])
                hm = (yv > hyv - 1.25) & (yv < hyv + 0.75)
                p0 = xv - hxv
                p1 = yv - hyv
                p2 = zv - hzv
                pxv = cv * p0 + sv * p2
                pzv = (-sv) * p0 + cv * p2
                rf = (pzv + 12.0) / 0.1
                cf = (pxv + 12.0) / 0.1
                ri = ((rf + _MAGIC) - _MAGIC).astype(jnp.int32)
                ci = ((cf + _MAGIC) - _MAGIC).astype(jnp.int32)
                ok = (hm & (ri >= 0) & (ri < _NUM_ROWS)
                      & (ci >= 0) & (ci < _NUM_COLS))
                ric = jnp.clip(ri, 0, _NUM_ROWS - 1)
                cic = jnp.clip(ci, 0, _NUM_COLS - 1)
                gib[sl] = (biv * _MAPB + sev * (_NUM_ROWS * _NUM_COLS)
                           + ric * _NUM_COLS + cic)
                gvb[sl] = jnp.where(ok, p1, 0.0)
            return 0
        lax.fori_loop(0, _K // 80, vbody, 0)

    def a_step(jj, _):
        for s in range(2):
            c = jj * 2 + s

            @pl.when(jj > 0)
            def _():
                a_outs(c - 2, s, False)
            a_loads(c, s, False)
            a_compute(s)
            a_outs(c, s, True)

            @pl.when(jj < _NCHUNK // 2 - 1)
            def _():
                a_loads(c + 2, s, True)
        return 0
    if _ABLATE_A:
        pass
    else:
        with jax.named_scope("phaseA"):
            lax.fori_loop(0, _NCHUNK // 2, a_step, 0)
            a_outs(_NCHUNK - 2, 0, False)
            a_outs(_NCHUNK - 1, 1, False)
    plsc.subcore_barrier()

    def b_loads(c, s, issue):
        gib, _ = pair_slots[s]
        xb, yb = in_slots[s][0], in_slots[s][1]
        base = goff + c * _K
        for src, dst in ((gidx.at[pl.ds(base, _K)], gib),
                         (gval.at[pl.ds(base, _K)], yb)):
            d = pltpu.make_async_copy(src, dst, in_slots[s][5])
            d.start() if issue else d.wait()

    def b_scatter(s, issue):
        bib = in_slots[s][3]
        gvb = pair_slots[s][1]
        d = pltpu.make_async_copy(gvb, smap.at[bib], sc_sems[s])
        if issue:
            pltpu.async_copy(gvb, smap.at[bib], sc_sems[s], add=True)
        else:
            d.wait()

    def b_compute(s, b):
        gib = pair_slots[s][0]
        yb = in_slots[s][1]
        bib = in_slots[s][3]
        gvb = pair_slots[s][1]
        lo = b * _MAPB

        def vbody(v, _):
            for u in range(5):
                sl = pl.ds((v * 5 + u) * 16, 16)
                dlt = gib[sl] - lo
                ok = (dlt >= 0) & (dlt < _MAPB)
                bib[sl] = jnp.clip(dlt, 0, _MAPB - 1)
                gvb[sl] = jnp.where(ok, yb[sl], 0.0)
            return 0
        lax.fori_loop(0, _K // 80, vbody, 0)

    def pass_body(k, _):
        b = cid * 8 + k

        with jax.named_scope("zero"):
            def zfill(i, _):
                buf0[pl.ds(i * 16, 16)] = jnp.zeros((16,), jnp.float32)
                return 0
            lax.fori_loop(0, _ZB // 16, zfill, 0)
            for q in range(_NSUB):
                pltpu.async_copy(
                    buf0, smap.at[pl.ds(sid * _SLICE + q * _ZB, _ZB)], swb0)
            for q in range(_NSUB):
                pltpu.make_async_copy(
                    buf0, smap.at[pl.ds(sid * _SLICE + q * _ZB, _ZB)],
                    swb0).wait()
        plsc.subcore_barrier()

        def b_step(c, _):
            base = goff + c * _K
            pltpu.sync_copy(gidx.at[pl.ds(base, _K)], gib0)
            pltpu.sync_copy(gval.at[pl.ds(base, _K)], yb0)
            b_compute(0, b)
            pltpu.sync_copy(gvb0, smap.at[bib0], add=True)
            return 0
        if not _ABLATE_SCAN:
            with jax.named_scope("scan"):
                lax.fori_loop(0, _NCHUNK, b_step, 0)
        plsc.subcore_barrier()

        with jax.named_scope("writeback"):
            for q in range(_NSUB):
                bq, sq = wb[q % 2]
                if q >= 2:
                    pltpu.make_async_copy(
                        bq, out.at[pl.ds(b * _MAPB, _ZB)], sq).wait()
                pltpu.sync_copy(
                    smap.at[pl.ds(sid * _SLICE + q * _ZB, _ZB)], bq)
                pltpu.async_copy(
                    bq, out.at[pl.ds(b * _MAPB + sid * _SLICE + q * _ZB, _ZB)],
                    sq)
            for q in (_NSUB - 2, _NSUB - 1):
                bq, sq = wb[q % 2]
                pltpu.make_async_copy(
                    bq, out.at[pl.ds(b * _MAPB, _ZB)], sq).wait()
        return 0
    lax.fori_loop(0, 8, pass_body, 0)


_sc_call = pl.kernel(
    _body,
    out_type=(jax.ShapeDtypeStruct((_B * _MAPB,), jnp.float32),
              jax.ShapeDtypeStruct((2 * _NP,), jnp.int32),
              jax.ShapeDtypeStruct((2 * _NP,), jnp.float32)),
    mesh=plsc.VectorSubcoreMesh(core_axis_name="c", subcore_axis_name="s",
                                num_cores=2, num_subcores=16),
    compiler_params=pltpu.CompilerParams(needs_layout_passes=False),
    scratch_types=[
        pltpu.VMEM((16,), jnp.float32),
        pltpu.VMEM((16,), jnp.float32),
        pltpu.VMEM((16,), jnp.float32),
        pltpu.VMEM((16,), jnp.float32),
        pltpu.VMEM((16,), jnp.float32),
        pltpu.VMEM((_K,), jnp.float32),
        pltpu.VMEM((_K,), jnp.float32),
        pltpu.VMEM((_K,), jnp.float32),
        pltpu.VMEM((_K,), jnp.int32),
        pltpu.VMEM((_K,), jnp.int32),
        pltpu.VMEM((_K,), jnp.int32),
        pltpu.VMEM((_K,), jnp.float32),
        pltpu.VMEM((_K,), jnp.float32),
        pltpu.VMEM((_K,), jnp.float32),
        pltpu.VMEM((_K,), jnp.float32),
        pltpu.VMEM((_K,), jnp.int32),
        pltpu.VMEM((_K,), jnp.int32),
        pltpu.VMEM((_K,), jnp.int32),
        pltpu.VMEM((_K,), jnp.float32),
        pltpu.VMEM((_ZB,), jnp.float32),
        pltpu.VMEM((_ZB,), jnp.float32),
        pltpu.VMEM_SHARED((_MAPB,), jnp.float32),
        pltpu.SemaphoreType.DMA,
        pltpu.SemaphoreType.DMA,
        pltpu.SemaphoreType.DMA,
        pltpu.SemaphoreType.DMA,
        pltpu.SemaphoreType.DMA,
        pltpu.SemaphoreType.DMA,
        pltpu.SemaphoreType.DMA,
        pltpu.SemaphoreType.DMA,
    ],
)


def kernel(xyz, batch_indices, semantics, robot_pose, robot_heading):
    pad = _NP - _N
    xt = jnp.pad(jnp.transpose(xyz), ((0, 0), (0, pad)))
    bi = jnp.pad(batch_indices.astype(jnp.int32), (0, pad),
                 constant_values=_B)
    se = jnp.pad(semantics.astype(jnp.int32), (0, pad))
    ang = -robot_heading
    out, _, _ = _sc_call(xt[0], xt[1], xt[2], bi, se,
                         robot_pose[:, 0], robot_pose[:, 1], robot_pose[:, 2],
                         jnp.cos(ang), jnp.sin(ang))
    return out.reshape(_B, _NUM_CLASSES, _NUM_ROWS, _NUM_COLS)


# speedup vs baseline: 14.3806x; 14.3806x over previous
import jax
import jax.numpy as jnp
from jax import lax
from jax.experimental import pallas as pl
from jax.experimental.pallas import tpu as pltpu
from jax.experimental.pallas import tpu_sc as plsc

_B = 16
_NUM_CLASSES = 20
_NUM_ROWS = 240
_NUM_COLS = 240
_MAPB = _NUM_CLASSES * _NUM_ROWS * _NUM_COLS
_N = 1_000_000
_NP = 1_024_000
_SPAN = _NP // 16
_K = 2_000
_NCHUNK = _SPAN // _K
_SLICE = _MAPB // 16
_ZB = 7_200
_NSUB = _SLICE // _ZB
_MAGIC = 12582912.0


def _body(xs, ys, zs, bi, se, hx, hy, hz, cs, sn,
          out, gidx, gval,
          thx, thy, thz, tc, ts,
          xb0, yb0, zb0, bib0, seb0, gib0, gvb0,
          xb1, yb1, zb1, bib1, seb1, gib1, gvb1,
          buf0, buf1, smap,
          sin0, sin1, sout0, sout1, ssc0, ssc1, swb0, swb1):
    cid = lax.axis_index("c")
    sid = lax.axis_index("s")
    goff = cid * _NP + sid * _SPAN

    in_slots = ((xb0, yb0, zb0, bib0, seb0, sin0),
                (xb1, yb1, zb1, bib1, seb1, sin1))
    pair_slots = ((gib0, gvb0), (gib1, gvb1))
    out_sems = (sout0, sout1)
    sc_sems = (ssc0, ssc1)
    wb = ((buf0, swb0), (buf1, swb1))

    def a_loads(c, s, issue):
        xb, yb, zb, bib, seb, sem = in_slots[s]
        base = sid * _SPAN + c * _K
        for src, dst in ((xs, xb), (ys, yb), (zs, zb), (bi, bib), (se, seb)):
            d = pltpu.make_async_copy(src.at[pl.ds(base, _K)], dst, sem)
            d.start() if issue else d.wait()

    def a_outs(c, s, issue):
        gib, gvb = pair_slots[s]
        base = goff + c * _K
        for src, dst in ((gib, gidx.at[pl.ds(base, _K)]),
                         (gvb, gval.at[pl.ds(base, _K)])):
            d = pltpu.make_async_copy(src, dst, out_sems[s])
            d.start() if issue else d.wait()

    pltpu.sync_copy(hx, thx)
    pltpu.sync_copy(hy, thy)
    pltpu.sync_copy(hz, thz)
    pltpu.sync_copy(cs, tc)
    pltpu.sync_copy(sn, ts)
    a_loads(0, 0, True)
    a_loads(1, 1, True)

    def a_compute(s):
        xb, yb, zb, bib, seb, _ = in_slots[s]
        gib, gvb = pair_slots[s]

        def vbody(v, _):
            for u in range(5):
                sl = pl.ds((v * 5 + u) * 16, 16)
                xv = xb[sl]
                yv = yb[sl]
                zv = zb[sl]
                biv = bib[sl]
                sev = seb[sl]
                hxv = plsc.load_gather(thx, [biv])
                hyv = plsc.load_gather(thy, [biv])
                hzv = plsc.load_gather(thz, [biv])
                cv = plsc.load_gather(tc, [biv])
                sv = plsc.load_gather(ts, [biv])
                hm = (yv > hyv - 1.25) & (yv < hyv + 0.75)
                p0 = xv - hxv
                p1 = yv - hyv
                p2 = zv - hzv
                pxv = cv * p0 + sv * p2
                pzv = (-sv) * p0 + cv * p2
                rf = (pzv + 12.0) / 0.1
                cf = (pxv + 12.0) / 0.1
                ri = ((rf + _MAGIC) - _MAGIC).astype(jnp.int32)
                ci = ((cf + _MAGIC) - _MAGIC).astype(jnp.int32)
                ok = (hm & (ri >= 0) & (ri < _NUM_ROWS)
                      & (ci >= 0) & (ci < _NUM_COLS))
                ric = jnp.clip(ri, 0, _NUM_ROWS - 1)
                cic = jnp.clip(ci, 0, _NUM_COLS - 1)
                gib[sl] = ((biv << 21) | (sev * (_NUM_ROWS * _NUM_COLS)
                                          + ric * _NUM_COLS + cic))
                gvb[sl] = jnp.where(ok, p1, 0.0)
            return 0
        lax.fori_loop(0, _K // 80, vbody, 0)

    def a_step(jj, _):
        for s in range(2):
            c = jj * 2 + s

            @pl.when(jj > 0)
            def _():
                a_outs(c - 2, s, False)
            a_loads(c, s, False)
            a_compute(s)
            a_outs(c, s, True)

            @pl.when(jj < _NCHUNK // 2 - 1)
            def _():
                a_loads(c + 2, s, True)
        return 0
    with jax.named_scope("phaseA"):
        lax.fori_loop(0, _NCHUNK // 2, a_step, 0)
        a_outs(_NCHUNK - 2, 0, False)
        a_outs(_NCHUNK - 1, 1, False)
    plsc.subcore_barrier()

    def b_loads(c, s, issue):
        gib, _ = pair_slots[s]
        xb, yb = in_slots[s][0], in_slots[s][1]
        base = goff + c * _K
        for src, dst in ((gidx.at[pl.ds(base, _K)], gib),
                         (gval.at[pl.ds(base, _K)], yb)):
            d = pltpu.make_async_copy(src, dst, in_slots[s][5])
            d.start() if issue else d.wait()

    def b_scatter(s, issue):
        bib = in_slots[s][3]
        gvb = pair_slots[s][1]
        d = pltpu.make_async_copy(gvb, smap.at[bib], sc_sems[s])
        if issue:
            pltpu.async_copy(gvb, smap.at[bib], sc_sems[s], add=True)
        else:
            d.wait()

    def b_compute(s, b):
        gib = pair_slots[s][0]
        yb = in_slots[s][1]
        bib = in_slots[s][3]
        gvb = pair_slots[s][1]
        def vbody(v, _):
            for u in range(5):
                sl = pl.ds((v * 5 + u) * 16, 16)
                g = gib[sl]
                ok = (g >> 21) == b
                bib[sl] = g & 0x1FFFFF
                gvb[sl] = jnp.where(ok, yb[sl], 0.0)
            return 0
        lax.fori_loop(0, _K // 80, vbody, 0)

    def pass_body(k, _):
        b = cid * 8 + k

        with jax.named_scope("zero"):
            def zfill(i, _):
                buf0[pl.ds(i * 16, 16)] = jnp.zeros((16,), jnp.float32)
                return 0
            lax.fori_loop(0, _ZB // 16, zfill, 0)
            for q in range(_NSUB):
                pltpu.async_copy(
                    buf0, smap.at[pl.ds(sid * _SLICE + q * _ZB, _ZB)], swb0)
            for q in range(_NSUB):
                pltpu.make_async_copy(
                    buf0, smap.at[pl.ds(sid * _SLICE + q * _ZB, _ZB)],
                    swb0).wait()
        plsc.subcore_barrier()

        b_loads(0, 0, True)
        b_loads(1, 1, True)

        def b_step(jj, _):
            for s in range(2):
                c = jj * 2 + s

                @pl.when(jj > 0)
                def _():
                    b_scatter(s, False)
                b_loads(c, s, False)
                b_compute(s, b)
                b_scatter(s, True)

                @pl.when(jj < _NCHUNK // 2 - 1)
                def _():
                    b_loads(c + 2, s, True)
            return 0
        with jax.named_scope("scan"):
            lax.fori_loop(0, _NCHUNK // 2, b_step, 0)
        b_scatter(0, False)
        b_scatter(1, False)
        plsc.subcore_barrier()

        with jax.named_scope("writeback"):
            for q in range(_NSUB):
                bq, sq = wb[q % 2]
                if q >= 2:
                    pltpu.make_async_copy(
                        bq, out.at[pl.ds(b * _MAPB, _ZB)], sq).wait()
                pltpu.sync_copy(
                    smap.at[pl.ds(sid * _SLICE + q * _ZB, _ZB)], bq)
                pltpu.async_copy(
                    bq, out.at[pl.ds(b * _MAPB + sid * _SLICE + q * _ZB, _ZB)],
                    sq)
            for q in (_NSUB - 2, _NSUB - 1):
                bq, sq = wb[q % 2]
                pltpu.make_async_copy(
                    bq, out.at[pl.ds(b * _MAPB, _ZB)], sq).wait()
        return 0
    lax.fori_loop(0, 8, pass_body, 0)


_sc_call = pl.kernel(
    _body,
    out_type=(jax.ShapeDtypeStruct((_B * _MAPB,), jnp.float32),
              jax.ShapeDtypeStruct((2 * _NP,), jnp.int32),
              jax.ShapeDtypeStruct((2 * _NP,), jnp.float32)),
    mesh=plsc.VectorSubcoreMesh(core_axis_name="c", subcore_axis_name="s",
                                num_cores=2, num_subcores=16),
    compiler_params=pltpu.CompilerParams(needs_layout_passes=False),
    scratch_types=[
        pltpu.VMEM((16,), jnp.float32),
        pltpu.VMEM((16,), jnp.float32),
        pltpu.VMEM((16,), jnp.float32),
        pltpu.VMEM((16,), jnp.float32),
        pltpu.VMEM((16,), jnp.float32),
        pltpu.VMEM((_K,), jnp.float32),
        pltpu.VMEM((_K,), jnp.float32),
        pltpu.VMEM((_K,), jnp.float32),
        pltpu.VMEM((_K,), jnp.int32),
        pltpu.VMEM((_K,), jnp.int32),
        pltpu.VMEM((_K,), jnp.int32),
        pltpu.VMEM((_K,), jnp.float32),
        pltpu.VMEM((_K,), jnp.float32),
        pltpu.VMEM((_K,), jnp.float32),
        pltpu.VMEM((_K,), jnp.float32),
        pltpu.VMEM((_K,), jnp.int32),
        pltpu.VMEM((_K,), jnp.int32),
        pltpu.VMEM((_K,), jnp.int32),
        pltpu.VMEM((_K,), jnp.float32),
        pltpu.VMEM((_ZB,), jnp.float32),
        pltpu.VMEM((_ZB,), jnp.float32),
        pltpu.VMEM_SHARED((_MAPB,), jnp.float32),
        pltpu.SemaphoreType.DMA,
        pltpu.SemaphoreType.DMA,
        pltpu.SemaphoreType.DMA,
        pltpu.SemaphoreType.DMA,
        pltpu.SemaphoreType.DMA,
        pltpu.SemaphoreType.DMA,
        pltpu.SemaphoreType.DMA,
        pltpu.SemaphoreType.DMA,
    ],
)


def kernel(xyz, batch_indices, semantics, robot_pose, robot_heading):
    pad = _NP - _N
    xt = jnp.pad(jnp.transpose(xyz), ((0, 0), (0, pad)))
    bi = jnp.pad(batch_indices.astype(jnp.int32), (0, pad),
                 constant_values=_B)
    se = jnp.pad(semantics.astype(jnp.int32), (0, pad))
    ang = -robot_heading
    out, _, _ = _sc_call(xt[0], xt[1], xt[2], bi, se,
                         robot_pose[:, 0], robot_pose[:, 1], robot_pose[:, 2],
                         jnp.cos(ang), jnp.sin(ang))
    return out.reshape(_B, _NUM_CLASSES, _NUM_ROWS, _NUM_COLS)
